# baseline (device time: 73847 ns/iter reference)
import jax
import jax.numpy as jnp
from jax import lax
from jax.experimental import pallas as pl
from jax.experimental.pallas import tpu as pltpu

N_DEV = 8
N_CHUNK = 8
DH = 64


def _ring_pos(p):
    return jnp.where(p < 4, p, 11 - p)


def _ring_dev(r):
    return jnp.where(r < 4, r, 11 - r)


def _allreduce_body(p_ref, out_ref, recv_ref, send_sems, recv_sems):
    my = lax.axis_index("i")
    pos = _ring_pos(my)
    right = _ring_dev((pos + 1) % N_DEV)
    left = _ring_dev((pos - 1) % N_DEV)

    out_ref[...] = p_ref[...]

    barrier_sem = pltpu.get_barrier_semaphore()
    pl.semaphore_signal(
        barrier_sem, inc=1, device_id=(left,), device_id_type=pl.DeviceIdType.MESH
    )
    pl.semaphore_signal(
        barrier_sem, inc=1, device_id=(right,), device_id_type=pl.DeviceIdType.MESH
    )
    pl.semaphore_wait(barrier_sem, 2)

    rows = out_ref.shape[0] // N_CHUNK

    for h in range(N_DEV - 1):
        c_send = ((pos - h) % N_CHUNK) * rows
        c_recv = ((pos - h - 1) % N_CHUNK) * rows
        rdma = pltpu.make_async_remote_copy(
            src_ref=out_ref.at[pl.ds(c_send, rows), :],
            dst_ref=recv_ref.at[h],
            send_sem=send_sems.at[h],
            recv_sem=recv_sems.at[h],
            device_id=(right,),
            device_id_type=pl.DeviceIdType.MESH,
        )
        rdma.start()
        rdma.wait()
        out_ref[pl.ds(c_recv, rows), :] += recv_ref[h]

    for g in range(N_DEV - 1):
        h = N_DEV - 1 + g
        c_send = ((pos + 1 - g) % N_CHUNK) * rows
        c_recv = ((pos - g) % N_CHUNK) * rows
        rdma = pltpu.make_async_remote_copy(
            src_ref=out_ref.at[pl.ds(c_send, rows), :],
            dst_ref=recv_ref.at[h],
            send_sem=send_sems.at[h],
            recv_sem=recv_sems.at[h],
            device_id=(right,),
            device_id_type=pl.DeviceIdType.MESH,
        )
        rdma.start()
        rdma.wait()
        out_ref[pl.ds(c_recv, rows), :] = recv_ref[h]


def _all_reduce(partial):
    m, n = partial.shape
    n_hops = 2 * (N_DEV - 1)
    return pl.pallas_call(
        _allreduce_body,
        out_shape=jax.ShapeDtypeStruct((m, n), partial.dtype),
        in_specs=[pl.BlockSpec(memory_space=pltpu.VMEM)],
        out_specs=pl.BlockSpec(memory_space=pltpu.VMEM),
        scratch_shapes=[
            pltpu.VMEM((n_hops, m // N_CHUNK, n), partial.dtype),
            pltpu.SemaphoreType.DMA((n_hops,)),
            pltpu.SemaphoreType.DMA((n_hops,)),
        ],
        compiler_params=pltpu.CompilerParams(collective_id=0),
    )(partial)


def kernel(x, Wq, Wo, K_ext, V_ext):
    B, Sq, D = x.shape
    Hl = Wq.shape[1] // DH
    my = lax.axis_index("i")

    xb = x.astype(jnp.bfloat16)
    Q = (xb.reshape(B * Sq, D) @ Wq.astype(jnp.bfloat16)).reshape(B, Sq, Hl, DH)
    K = lax.dynamic_slice_in_dim(K_ext, my * Hl, Hl, axis=2).astype(jnp.bfloat16)
    V = lax.dynamic_slice_in_dim(V_ext, my * Hl, Hl, axis=2).astype(jnp.bfloat16)

    s = jnp.einsum("bqhd,bkhd->bhqk", Q, K, preferred_element_type=jnp.float32)
    p = jax.nn.softmax(s * 0.125, axis=-1).astype(jnp.bfloat16)
    o = jnp.einsum("bhqk,bkhd->bqhd", p, V, preferred_element_type=jnp.float32)

    partial = jnp.dot(
        o.reshape(B * Sq, Hl * DH).astype(jnp.bfloat16),
        Wo.astype(jnp.bfloat16),
        preferred_element_type=jnp.float32,
    )

    out = _all_reduce(partial)
    return out.reshape(B, Sq, D)


# device time: 46354 ns/iter; 1.5931x vs baseline; 1.5931x over previous
import jax
import jax.numpy as jnp
from jax import lax
from jax.experimental import pallas as pl
from jax.experimental.pallas import tpu as pltpu

N_DEV = 8
N_CHUNK = 8
DH = 64

_MESH = pl.DeviceIdType.MESH


def _gray(v):
    v4 = v & 3
    return (v4 ^ (v4 >> 1)) | (v & 4)


def _butterfly_body(p_ref, out_ref, s_ref, r_ref, w_ref, send_sems, recv_sems):
    my = lax.axis_index("i")
    q = _gray(my)

    out_ref[...] = p_ref[...]

    barrier_sem = pltpu.get_barrier_semaphore()
    for k in range(3):
        partner = _gray(q ^ (1 << k))
        pl.semaphore_signal(
            barrier_sem, inc=1, device_id=(partner,), device_id_type=_MESH
        )
    pl.semaphore_wait(barrier_sem, 3)

    offs = (0, 4, 6)

    base = jnp.int32(0)
    size = N_CHUNK
    for k in range(3):
        half = size // 2
        bit = (q >> k) & 1
        partner = _gray(q ^ (1 << k))
        send_off = jnp.where(bit == 1, base, base + half)
        keep_off = jnp.where(bit == 1, base + half, base)
        s_ref[offs[k] : offs[k] + half] = out_ref[pl.ds(send_off, half)].astype(
            jnp.bfloat16
        )
        rdma = pltpu.make_async_remote_copy(
            src_ref=s_ref.at[offs[k] : offs[k] + half],
            dst_ref=r_ref.at[offs[k] : offs[k] + half],
            send_sem=send_sems.at[k],
            recv_sem=recv_sems.at[k],
            device_id=(partner,),
            device_id_type=_MESH,
        )
        rdma.start()
        rdma.wait()
        out_ref[pl.ds(keep_off, half)] += r_ref[
            offs[k] : offs[k] + half
        ].astype(jnp.float32)
        base = keep_off
        size = half

    w_ref[pl.ds(base, 1)] = out_ref[pl.ds(base, 1)].astype(jnp.bfloat16)

    for j in range(3):
        k = 2 - j
        bit = (q >> k) & 1
        partner = _gray(q ^ (1 << k))
        pbase = jnp.where(bit == 1, base - size, base + size)
        rdma = pltpu.make_async_remote_copy(
            src_ref=w_ref.at[pl.ds(base, size)],
            dst_ref=w_ref.at[pl.ds(base, size)],
            send_sem=send_sems.at[3 + j],
            recv_sem=recv_sems.at[3 + j],
            device_id=(partner,),
            device_id_type=_MESH,
        )
        rdma.start()
        rdma.wait()
        out_ref[pl.ds(pbase, size)] = w_ref[pl.ds(pbase, size)].astype(
            jnp.float32
        )
        base = jnp.minimum(base, pbase)
        size = 2 * size


def _all_reduce(partial):
    nc, rows, n = partial.shape
    return pl.pallas_call(
        _butterfly_body,
        out_shape=jax.ShapeDtypeStruct((nc, rows, n), partial.dtype),
        in_specs=[pl.BlockSpec(memory_space=pltpu.VMEM)],
        out_specs=pl.BlockSpec(memory_space=pltpu.VMEM),
        scratch_shapes=[
            pltpu.VMEM((7, rows, n), jnp.bfloat16),
            pltpu.VMEM((7, rows, n), jnp.bfloat16),
            pltpu.VMEM((nc, rows, n), jnp.bfloat16),
            pltpu.SemaphoreType.DMA((6,)),
            pltpu.SemaphoreType.DMA((6,)),
        ],
        compiler_params=pltpu.CompilerParams(collective_id=0),
    )(partial)


def kernel(x, Wq, Wo, K_ext, V_ext):
    B, Sq, D = x.shape
    Hl = Wq.shape[1] // DH
    my = lax.axis_index("i")

    xb = x.astype(jnp.bfloat16)
    Q = (xb.reshape(B * Sq, D) @ Wq.astype(jnp.bfloat16)).reshape(B, Sq, Hl, DH)
    K = lax.dynamic_slice_in_dim(K_ext, my * Hl, Hl, axis=2).astype(jnp.bfloat16)
    V = lax.dynamic_slice_in_dim(V_ext, my * Hl, Hl, axis=2).astype(jnp.bfloat16)

    s = jnp.einsum("bqhd,bkhd->bhqk", Q, K, preferred_element_type=jnp.float32)
    p = jax.nn.softmax(s * 0.125, axis=-1).astype(jnp.bfloat16)
    o = jnp.einsum("bhqk,bkhd->bqhd", p, V, preferred_element_type=jnp.float32)

    partial = jnp.dot(
        o.reshape(B * Sq, Hl * DH).astype(jnp.bfloat16),
        Wo.astype(jnp.bfloat16),
        preferred_element_type=jnp.float32,
    )

    rows = (B * Sq) // N_CHUNK
    out = _all_reduce(partial.reshape(N_CHUNK, rows, D))
    return out.reshape(B, Sq, D)


# device time: 36035 ns/iter; 2.0493x vs baseline; 1.2864x over previous
import jax
import jax.numpy as jnp
from jax import lax
from jax.experimental import pallas as pl
from jax.experimental.pallas import tpu as pltpu

N_DEV = 8
N_CHUNK = 8
DH = 64

_MESH = pl.DeviceIdType.MESH


def _fused_body(
    o_ref,
    wo_ref,
    out_ref,
    s_ref,
    r_ref,
    ag_src,
    ag_ref,
    rs_send,
    rs_recv,
    ag_send,
    ag_recv,
):
    my = lax.axis_index("i")

    barrier_sem = pltpu.get_barrier_semaphore()
    for t in range(N_DEV - 1):
        peer = (my + 1 + t) % N_DEV
        pl.semaphore_signal(
            barrier_sem, inc=1, device_id=(peer,), device_id_type=_MESH
        )
    pl.semaphore_wait(barrier_sem, N_DEV - 1)

    wo = wo_ref[...]

    rs = []
    for t in range(N_DEV - 1):
        c = (my + 1 + t) % N_CHUNK
        part = jnp.dot(
            o_ref[pl.ds(c, 1)][0], wo, preferred_element_type=jnp.float32
        )
        s_ref[t] = part.astype(jnp.bfloat16)
        rdma = pltpu.make_async_remote_copy(
            src_ref=s_ref.at[t],
            dst_ref=r_ref.at[t],
            send_sem=rs_send.at[t],
            recv_sem=rs_recv.at[t],
            device_id=(c,),
            device_id_type=_MESH,
        )
        rdma.start()
        rs.append(rdma)

    own = jnp.dot(
        o_ref[pl.ds(my, 1)][0], wo, preferred_element_type=jnp.float32
    )

    for t in range(N_DEV - 1):
        rs[t].wait_recv()
    red = own + jnp.sum(r_ref[...].astype(jnp.float32), axis=0)
    out_ref[pl.ds(my, 1)] = red[None]
    ag_src[...] = red.astype(jnp.bfloat16)

    ags = []
    for t in range(N_DEV - 1):
        c = (my + 1 + t) % N_CHUNK
        rdma = pltpu.make_async_remote_copy(
            src_ref=ag_src,
            dst_ref=ag_ref.at[t],
            send_sem=ag_send.at[t],
            recv_sem=ag_recv.at[t],
            device_id=(c,),
            device_id_type=_MESH,
        )
        rdma.start()
        ags.append(rdma)

    for t in range(N_DEV - 1):
        rs[t].wait_send()
    for t in range(N_DEV - 1):
        ags[t].wait_recv()
        c = (my + N_DEV - 1 - t) % N_CHUNK
        out_ref[pl.ds(c, 1)] = ag_ref[t][None].astype(jnp.float32)
    for t in range(N_DEV - 1):
        ags[t].wait_send()


def _matmul_all_reduce(o, Wo):
    nc, rows, hd = o.shape
    n = Wo.shape[1]
    return pl.pallas_call(
        _fused_body,
        out_shape=jax.ShapeDtypeStruct((nc, rows, n), jnp.float32),
        in_specs=[
            pl.BlockSpec(memory_space=pltpu.VMEM),
            pl.BlockSpec(memory_space=pltpu.VMEM),
        ],
        out_specs=pl.BlockSpec(memory_space=pltpu.VMEM),
        scratch_shapes=[
            pltpu.VMEM((N_DEV - 1, rows, n), jnp.bfloat16),
            pltpu.VMEM((N_DEV - 1, rows, n), jnp.bfloat16),
            pltpu.VMEM((rows, n), jnp.bfloat16),
            pltpu.VMEM((N_DEV - 1, rows, n), jnp.bfloat16),
            pltpu.SemaphoreType.DMA((N_DEV - 1,)),
            pltpu.SemaphoreType.DMA((N_DEV - 1,)),
            pltpu.SemaphoreType.DMA((N_DEV - 1,)),
            pltpu.SemaphoreType.DMA((N_DEV - 1,)),
        ],
        compiler_params=pltpu.CompilerParams(collective_id=0),
    )(o, Wo)


def kernel(x, Wq, Wo, K_ext, V_ext):
    B, Sq, D = x.shape
    Hl = Wq.shape[1] // DH
    my = lax.axis_index("i")

    xb = x.astype(jnp.bfloat16)
    Q = (xb.reshape(B * Sq, D) @ Wq.astype(jnp.bfloat16)).reshape(B, Sq, Hl, DH)
    K = lax.dynamic_slice_in_dim(K_ext, my * Hl, Hl, axis=2).astype(jnp.bfloat16)
    V = lax.dynamic_slice_in_dim(V_ext, my * Hl, Hl, axis=2).astype(jnp.bfloat16)

    s = jnp.einsum("bqhd,bkhd->bhqk", Q, K, preferred_element_type=jnp.float32)
    p = jax.nn.softmax(s * 0.125, axis=-1).astype(jnp.bfloat16)
    o = jnp.einsum("bhqk,bkhd->bqhd", p, V, preferred_element_type=jnp.float32)

    rows = (B * Sq) // N_CHUNK
    o_chunks = o.reshape(N_CHUNK, rows, Hl * DH).astype(jnp.bfloat16)
    out = _matmul_all_reduce(o_chunks, Wo.astype(jnp.bfloat16))
    return out.reshape(B, Sq, D)


# device time: 29695 ns/iter; 2.4868x vs baseline; 1.2135x over previous
import jax
import jax.numpy as jnp
from jax import lax
from jax.experimental import pallas as pl
from jax.experimental.pallas import tpu as pltpu

N_DEV = 8
N_CHUNK = 8
DH = 64
B = 2
SQ = 256
ROWS = (B * SQ) // N_CHUNK

_MESH = pl.DeviceIdType.MESH


def _fused_body(
    x_ref,
    wq_ref,
    k_ref,
    v_ref,
    wo_ref,
    out_ref,
    q_ref,
    o_ref,
    s_ref,
    r_ref,
    rs_send,
    rs_recv,
    ag_send,
    ag_recv,
):
    my = lax.axis_index("i")
    n_heads = k_ref.shape[1]

    barrier_sem = pltpu.get_barrier_semaphore()
    for t in range(N_DEV - 1):
        peer = (my + 1 + t) % N_DEV
        pl.semaphore_signal(
            barrier_sem, inc=1, device_id=(peer,), device_id_type=_MESH
        )

    q_ref[...] = jnp.dot(
        x_ref[...], wq_ref[...], preferred_element_type=jnp.float32
    ).astype(jnp.bfloat16)

    def attention_batch(b):
        for h in range(n_heads):
            q_bh = q_ref[b * SQ : (b + 1) * SQ, h * DH : (h + 1) * DH]
            s = lax.dot_general(
                q_bh,
                k_ref[b, h],
                (((1,), (1,)), ((), ())),
                preferred_element_type=jnp.float32,
            )
            m = jnp.max(s, axis=1, keepdims=True)
            p = jnp.exp(s * 0.125 - m * 0.125)
            l = jnp.sum(p, axis=1, keepdims=True)
            o = jnp.dot(
                p.astype(jnp.bfloat16),
                v_ref[b, h],
                preferred_element_type=jnp.float32,
            ) / l
            ob = o.astype(jnp.bfloat16)
            for i in range(SQ // ROWS):
                o_ref[4 * b + i, :, h * DH : (h + 1) * DH] = ob[
                    i * ROWS : (i + 1) * ROWS
                ]

    def send_chunks(chunks):
        for c in chunks:

            @pl.when(c != my)
            def _():
                part = jnp.dot(
                    o_ref[c], wo_ref[...], preferred_element_type=jnp.float32
                )
                s_ref[c] = part.astype(jnp.bfloat16)
                rdma = pltpu.make_async_remote_copy(
                    src_ref=s_ref.at[c],
                    dst_ref=r_ref.at[my],
                    send_sem=rs_send.at[c],
                    recv_sem=rs_recv.at[my],
                    device_id=(c,),
                    device_id_type=_MESH,
                )
                rdma.start()

    attention_batch(0)
    pl.semaphore_wait(barrier_sem, N_DEV - 1)
    send_chunks(range(4))
    attention_batch(1)
    send_chunks(range(4, 8))

    own = jnp.dot(
        o_ref[pl.ds(my, 1)][0], wo_ref[...], preferred_element_type=jnp.float32
    )
    r_ref[pl.ds(my, 1)] = own.astype(jnp.bfloat16)[None]

    for s_id in range(N_DEV):

        @pl.when(s_id != my)
        def _():
            recv = pltpu.make_async_remote_copy(
                src_ref=s_ref.at[s_id],
                dst_ref=r_ref.at[s_id],
                send_sem=rs_send.at[s_id],
                recv_sem=rs_recv.at[s_id],
                device_id=(my,),
                device_id_type=_MESH,
            )
            recv.wait_recv()

    red = jnp.sum(r_ref[...].astype(jnp.float32), axis=0)
    out_ref[pl.ds(my, 1)] = red.astype(jnp.bfloat16)[None]

    for c in range(N_DEV):

        @pl.when(c != my)
        def _():
            rdma = pltpu.make_async_remote_copy(
                src_ref=out_ref.at[pl.ds(my, 1)],
                dst_ref=out_ref.at[pl.ds(my, 1)],
                send_sem=ag_send.at[c],
                recv_sem=ag_recv.at[my],
                device_id=(c,),
                device_id_type=_MESH,
            )
            rdma.start()

    for c in range(N_DEV):

        @pl.when(c != my)
        def _():
            snd = pltpu.make_async_remote_copy(
                src_ref=s_ref.at[c],
                dst_ref=r_ref.at[c],
                send_sem=rs_send.at[c],
                recv_sem=rs_recv.at[c],
                device_id=(c,),
                device_id_type=_MESH,
            )
            snd.wait_send()

    for s_id in range(N_DEV):

        @pl.when(s_id != my)
        def _():
            recv = pltpu.make_async_remote_copy(
                src_ref=out_ref.at[pl.ds(s_id, 1)],
                dst_ref=out_ref.at[pl.ds(s_id, 1)],
                send_sem=ag_send.at[s_id],
                recv_sem=ag_recv.at[s_id],
                device_id=(my,),
                device_id_type=_MESH,
            )
            recv.wait_recv()

    for c in range(N_DEV):

        @pl.when(c != my)
        def _():
            snd = pltpu.make_async_remote_copy(
                src_ref=out_ref.at[pl.ds(my, 1)],
                dst_ref=out_ref.at[pl.ds(my, 1)],
                send_sem=ag_send.at[c],
                recv_sem=ag_recv.at[c],
                device_id=(c,),
                device_id_type=_MESH,
            )
            snd.wait_send()


def _fused_attention_all_reduce(xb, Wq, K, V, Wo):
    n = Wo.shape[1]
    return pl.pallas_call(
        _fused_body,
        out_shape=jax.ShapeDtypeStruct((N_CHUNK, ROWS, n), jnp.bfloat16),
        in_specs=[pl.BlockSpec(memory_space=pltpu.VMEM)] * 5,
        out_specs=pl.BlockSpec(memory_space=pltpu.VMEM),
        scratch_shapes=[
            pltpu.VMEM((B * SQ, Wq.shape[1]), jnp.bfloat16),
            pltpu.VMEM((N_CHUNK, ROWS, Wq.shape[1]), jnp.bfloat16),
            pltpu.VMEM((N_CHUNK, ROWS, n), jnp.bfloat16),
            pltpu.VMEM((N_CHUNK, ROWS, n), jnp.bfloat16),
            pltpu.SemaphoreType.DMA((N_DEV,)),
            pltpu.SemaphoreType.DMA((N_DEV,)),
            pltpu.SemaphoreType.DMA((N_DEV,)),
            pltpu.SemaphoreType.DMA((N_DEV,)),
        ],
        compiler_params=pltpu.CompilerParams(collective_id=0),
    )(xb, Wq, K, V, Wo)


def kernel(x, Wq, Wo, K_ext, V_ext):
    b, sq, d = x.shape
    Hl = Wq.shape[1] // DH
    my = lax.axis_index("i")

    xb = x.reshape(b * sq, d).astype(jnp.bfloat16)
    K = lax.dynamic_slice_in_dim(K_ext, my * Hl, Hl, axis=2)
    V = lax.dynamic_slice_in_dim(V_ext, my * Hl, Hl, axis=2)
    K = jnp.transpose(K, (0, 2, 1, 3)).astype(jnp.bfloat16)
    V = jnp.transpose(V, (0, 2, 1, 3)).astype(jnp.bfloat16)

    out = _fused_attention_all_reduce(
        xb, Wq.astype(jnp.bfloat16), K, V, Wo.astype(jnp.bfloat16)
    )
    return out.reshape(b, sq, d).astype(jnp.float32)
